# Initial kernel scaffold; baseline (speedup 1.0000x reference)
import jax
import jax.numpy as jnp
from jax.experimental import pallas as pl


def kernel(x, x_type, W, b):
    raise NotImplementedError("write your pallas kernel here")


# speedup vs baseline: 1.2386x; 1.2386x over previous
import functools

import jax
import jax.numpy as jnp
from jax import lax
from jax.experimental import pallas as pl
from jax.experimental.pallas import tpu as pltpu
from jax.experimental.pallas import tpu_sc as plsc

N = 2048
D = 768
T = 8
NC, NS, L = 2, 16, 16
NW = NC * NS
CHUNK = N // NW
BM = 256
NB = N // BM
G = NB + T - 1

_MESH = plsc.VectorSubcoreMesh(
    core_axis_name="c", subcore_axis_name="s", num_cores=NC, num_subcores=NS)


@functools.partial(
    pl.kernel,
    out_type=jax.ShapeDtypeStruct((NW, L), jnp.int32),
    mesh=_MESH,
    scratch_types=[
        pltpu.VMEM((CHUNK,), jnp.int32),
        pltpu.VMEM((L,), jnp.int32),
    ],
    compiler_params=pltpu.CompilerParams(needs_layout_passes=False),
)
def _hist_kernel(xt_hbm, hist_hbm, ty_v, hrow_v):
    w = lax.axis_index("s") * NC + lax.axis_index("c")
    base = w * CHUNK
    iota16 = lax.broadcasted_iota(jnp.int32, (L,), 0)

    pltpu.sync_copy(xt_hbm.at[pl.ds(base, CHUNK)], ty_v)

    h = jnp.zeros((L,), jnp.int32)
    for ci in range(CHUNK // L):
        tyc = ty_v[pl.ds(ci * L, L)]
        for t in range(T):
            pc = jnp.sum((tyc == t).astype(jnp.int32))
            h = jnp.where(iota16 == t, h + jnp.full((L,), pc), h)
    hrow_v[...] = h
    pltpu.sync_copy(hrow_v, hist_hbm.at[w])


@functools.partial(
    pl.kernel,
    out_type=(
        jax.ShapeDtypeStruct((N, D), jnp.float32),
        jax.ShapeDtypeStruct((N,), jnp.int32),
        jax.ShapeDtypeStruct((8, L), jnp.int32),
    ),
    mesh=_MESH,
    scratch_types=[
        pltpu.VMEM((CHUNK,), jnp.int32),
        pltpu.VMEM((CHUNK,), jnp.int32),
        pltpu.VMEM((CHUNK, D), jnp.float32),
        pltpu.VMEM((NW, L), jnp.int32),
        pltpu.VMEM((8, L), jnp.int32),
        pltpu.VMEM((24,), jnp.int32),
        pltpu.SemaphoreType.DMA,
    ],
    compiler_params=pltpu.CompilerParams(needs_layout_passes=False),
)
def _route_kernel(x_hbm, xt_hbm, hist_hbm, xs_hbm, pos_hbm, tbl_hbm,
                  ty_v, pos_v, rows_v, hall_v, tbl_v, bnd_v, sem):
    w = lax.axis_index("s") * NC + lax.axis_index("c")
    base = w * CHUNK
    iota16 = lax.broadcasted_iota(jnp.int32, (L,), 0)

    pltpu.sync_copy(xt_hbm.at[pl.ds(base, CHUNK)], ty_v)
    pltpu.sync_copy(hist_hbm, hall_v)

    total = jnp.zeros((L,), jnp.int32)
    mine = jnp.zeros((L,), jnp.int32)
    for wp in range(NW):
        row = hall_v[wp]
        flag = jnp.full((L,), (wp < w).astype(jnp.int32))
        mine = mine + row * flag
        total = total + row
    cs = plsc.cumsum(total)
    excl = cs - total
    base_vec = excl + mine

    for ci in range(CHUNK // L):
        tyc = ty_v[pl.ds(ci * L, L)]
        posc = jnp.zeros((L,), jnp.int32)
        for t in range(T):
            m = tyc == t
            cm = plsc.cumsum(m.astype(jnp.int32))
            bt = jnp.sum(jnp.where(iota16 == t, base_vec, 0))
            posc = jnp.where(m, jnp.full((L,), bt) + cm - 1, posc)
            cnt = jnp.max(cm)
            base_vec = jnp.where(iota16 == t, base_vec + jnp.full((L,), cnt),
                                 base_vec)
        posc = jnp.minimum(jnp.maximum(posc, jnp.zeros((L,), jnp.int32)),
                           jnp.full((L,), N - 1))
        pos_v[pl.ds(ci * L, L)] = posc

    pltpu.sync_copy(pos_v, pos_hbm.at[pl.ds(base, CHUNK)])
    pltpu.sync_copy(x_hbm.at[pl.ds(base, CHUNK)], rows_v)
    pltpu.async_copy(rows_v, xs_hbm.at[pos_v], sem).wait()

    @pl.when(w == 0)
    def _tables():
        off_hi = plsc.cumsum(total)
        bnd_v[pl.ds(0, L)] = iota16 * BM
        bnd_v[pl.ds(8, L)] = off_hi
        bvals = bnd_v[pl.ds(0, L)]
        startv = jnp.sort(bvals)
        bnd_v[pl.ds(8, L)] = jnp.full((L,), N, jnp.int32)
        bnd_v[pl.ds(0, L)] = startv
        endv = bnd_v[pl.ds(1, L)]
        typv = jnp.zeros((L,), jnp.int32)
        for t in range(T):
            ot = jnp.sum(jnp.where(iota16 == t, off_hi, 0))
            typv = typv + (startv >= jnp.full((L,), ot)).astype(jnp.int32)
        typv = jnp.minimum(typv, T - 1)
        blkv = jnp.minimum(lax.shift_right_logical(startv, 8), NB - 1)
        firstv = (((startv & (BM - 1)) == 0) & (startv < N)).astype(jnp.int32)
        tbl_v[0] = blkv
        tbl_v[1] = typv
        tbl_v[2] = startv
        tbl_v[3] = endv
        tbl_v[4] = firstv
        zeros = jnp.zeros((L,), jnp.int32)
        tbl_v[5] = zeros
        tbl_v[6] = zeros
        tbl_v[7] = zeros
        pltpu.sync_copy(tbl_v, tbl_hbm)


def _gemm_body(tbl_ref, x_ref, w_ref, b_ref, o_ref):
    g = pl.program_id(0)
    blk = tbl_ref[0, g]
    start = tbl_ref[2, g]
    end = tbl_ref[3, g]
    first = tbl_ref[4, g]
    s = start - blk * BM
    e = end - blk * BM

    @pl.when(first == 1)
    def _init():
        o_ref[...] = jnp.zeros_like(o_ref)

    @pl.when(e > s)
    def _acc():
        acc = jnp.dot(x_ref[...], w_ref[0],
                      preferred_element_type=jnp.float32)
        acc = acc + b_ref[0]
        ri = lax.broadcasted_iota(jnp.int32, (BM, 1), 0)
        m = (ri >= s) & (ri < e)
        o_ref[...] += jnp.where(m, acc, 0.0)


def _grouped_gemm(tbl, xs, W, b):
    grid_spec = pltpu.PrefetchScalarGridSpec(
        num_scalar_prefetch=1,
        grid=(G,),
        in_specs=[
            pl.BlockSpec((BM, D), lambda g, tbl: (tbl[0, g], 0)),
            pl.BlockSpec((1, D, D), lambda g, tbl: (tbl[1, g], 0, 0)),
            pl.BlockSpec((1, 1, D), lambda g, tbl: (tbl[1, g], 0, 0)),
        ],
        out_specs=pl.BlockSpec((BM, D), lambda g, tbl: (tbl[0, g], 0)),
    )
    return pl.pallas_call(
        _gemm_body,
        grid_spec=grid_spec,
        out_shape=jax.ShapeDtypeStruct((N, D), jnp.float32),
    )(tbl, xs, W, b.reshape(T, 1, D))


@functools.partial(
    pl.kernel,
    out_type=jax.ShapeDtypeStruct((N, D), jnp.float32),
    mesh=_MESH,
    scratch_types=[
        pltpu.VMEM((CHUNK,), jnp.int32),
        pltpu.VMEM((CHUNK, D), jnp.float32),
        pltpu.SemaphoreType.DMA,
    ],
    compiler_params=pltpu.CompilerParams(needs_layout_passes=False),
)
def _unsort_kernel(y_hbm, pos_hbm, out_hbm, pos_v, rows_v, sem):
    w = lax.axis_index("s") * NC + lax.axis_index("c")
    base = w * CHUNK
    pltpu.sync_copy(pos_hbm.at[pl.ds(base, CHUNK)], pos_v)
    pltpu.async_copy(y_hbm.at[pos_v], rows_v, sem).wait()
    pltpu.sync_copy(rows_v, out_hbm.at[pl.ds(base, CHUNK)])


@jax.jit
def kernel(x, x_type, W, b):
    xt = x_type.astype(jnp.int32)
    hist = _hist_kernel(xt)
    xs, pos, tbl = _route_kernel(x, xt, hist)
    y = _grouped_gemm(tbl, xs, W, b)
    return _unsort_kernel(y, pos)


# speedup vs baseline: 1.2414x; 1.0022x over previous
import functools

import jax
import jax.numpy as jnp
from jax import lax
from jax.experimental import pallas as pl
from jax.experimental.pallas import tpu as pltpu
from jax.experimental.pallas import tpu_sc as plsc

N = 2048
D = 768
T = 8
NC, NS, L = 2, 16, 16
NW = NC * NS
CHUNK = N // NW
BM = 128
NB = N // BM

_MESH = plsc.VectorSubcoreMesh(
    core_axis_name="c", subcore_axis_name="s", num_cores=NC, num_subcores=NS)


@functools.partial(
    pl.kernel,
    out_type=(
        jax.ShapeDtypeStruct((N, D), jnp.float32),
        jax.ShapeDtypeStruct((N,), jnp.int32),
        jax.ShapeDtypeStruct((L,), jnp.int32),
    ),
    mesh=_MESH,
    scratch_types=[
        pltpu.VMEM((N,), jnp.int32),
        pltpu.VMEM((CHUNK,), jnp.int32),
        pltpu.VMEM((CHUNK, D), jnp.float32),
        pltpu.VMEM((L,), jnp.int32),
        pltpu.SemaphoreType.DMA,
    ],
    compiler_params=pltpu.CompilerParams(needs_layout_passes=False),
)
def _route_kernel(x_hbm, xt_hbm, xs_hbm, pos_hbm, off_hbm,
                  ty_v, pos_v, rows_v, off_v, sem):
    w = lax.axis_index("s") * NC + lax.axis_index("c")
    base = w * CHUNK
    iota16 = lax.broadcasted_iota(jnp.int32, (L,), 0)

    pltpu.sync_copy(xt_hbm, ty_v)

    total = jnp.zeros((L,), jnp.int32)
    mine = jnp.zeros((L,), jnp.int32)
    for ci in range(NW):
        hc = jnp.zeros((L,), jnp.int32)
        for sub in range(CHUNK // L):
            tyc = ty_v[pl.ds(ci * CHUNK + sub * L, L)]
            for t in range(T):
                pc = jnp.sum((tyc == t).astype(jnp.int32))
                hc = jnp.where(iota16 == t, hc + jnp.full((L,), pc), hc)
        flag = jnp.full((L,), (ci < w).astype(jnp.int32))
        mine = mine + hc * flag
        total = total + hc
    cs = plsc.cumsum(total)
    excl = cs - total
    base_vec = excl + mine

    for ci in range(CHUNK // L):
        tyc = ty_v[pl.ds(base + ci * L, L)]
        posc = jnp.zeros((L,), jnp.int32)
        for t in range(T):
            m = tyc == t
            cm = plsc.cumsum(m.astype(jnp.int32))
            bt = jnp.sum(jnp.where(iota16 == t, base_vec, 0))
            posc = jnp.where(m, jnp.full((L,), bt) + cm - 1, posc)
            cnt = jnp.max(cm)
            base_vec = jnp.where(iota16 == t, base_vec + jnp.full((L,), cnt),
                                 base_vec)
        posc = jnp.minimum(jnp.maximum(posc, jnp.zeros((L,), jnp.int32)),
                           jnp.full((L,), N - 1))
        pos_v[pl.ds(ci * L, L)] = posc

    pltpu.sync_copy(pos_v, pos_hbm.at[pl.ds(base, CHUNK)])
    pltpu.sync_copy(x_hbm.at[pl.ds(base, CHUNK)], rows_v)
    pltpu.async_copy(rows_v, xs_hbm.at[pos_v], sem).wait()

    @pl.when(w == 0)
    def _offsets():
        off_v[...] = excl
        pltpu.sync_copy(off_v, off_hbm)


def _gemm_body(off_ref, x_ref, w_ref, b_ref, o_ref):
    blk = pl.program_id(0)
    lo = blk * BM
    ri = lax.broadcasted_iota(jnp.int32, (BM, 1), 0)
    acc = jnp.zeros((BM, D), jnp.float32)
    o_ref[...] = acc
    for t in range(T):
        s = off_ref[t] - lo
        e = off_ref[t + 1] - lo

        @pl.when((e > s) & (e > 0) & (s < BM))
        def _acc():
            prod = jnp.dot(x_ref[...], w_ref[t],
                           preferred_element_type=jnp.float32)
            prod = prod + b_ref[t][None, :]
            m = (ri >= s) & (ri < e)
            o_ref[...] += jnp.where(m, prod, 0.0)


def _grouped_gemm(offs, xs, W, b):
    grid_spec = pltpu.PrefetchScalarGridSpec(
        num_scalar_prefetch=1,
        grid=(NB,),
        in_specs=[
            pl.BlockSpec((BM, D), lambda g, off: (g, 0)),
            pl.BlockSpec((T, D, D), lambda g, off: (0, 0, 0)),
            pl.BlockSpec((T, D), lambda g, off: (0, 0)),
        ],
        out_specs=pl.BlockSpec((BM, D), lambda g, off: (g, 0)),
    )
    return pl.pallas_call(
        _gemm_body,
        grid_spec=grid_spec,
        out_shape=jax.ShapeDtypeStruct((N, D), jnp.float32),
    )(offs, xs, W, b)


@functools.partial(
    pl.kernel,
    out_type=jax.ShapeDtypeStruct((N, D), jnp.float32),
    mesh=_MESH,
    scratch_types=[
        pltpu.VMEM((CHUNK,), jnp.int32),
        pltpu.VMEM((CHUNK, D), jnp.float32),
        pltpu.SemaphoreType.DMA,
    ],
    compiler_params=pltpu.CompilerParams(needs_layout_passes=False),
)
def _unsort_kernel(y_hbm, pos_hbm, out_hbm, pos_v, rows_v, sem):
    w = lax.axis_index("s") * NC + lax.axis_index("c")
    base = w * CHUNK
    pltpu.sync_copy(pos_hbm.at[pl.ds(base, CHUNK)], pos_v)
    pltpu.async_copy(y_hbm.at[pos_v], rows_v, sem).wait()
    pltpu.sync_copy(rows_v, out_hbm.at[pl.ds(base, CHUNK)])


@jax.jit
def kernel(x, x_type, W, b):
    xt = x_type.astype(jnp.int32)
    xs, pos, offs = _route_kernel(x, xt)
    y = _grouped_gemm(offs, xs, W, b)
    return _unsort_kernel(y, pos)


# speedup vs baseline: 1.3685x; 1.1024x over previous
import functools

import jax
import jax.numpy as jnp
from jax import lax
from jax.experimental import pallas as pl
from jax.experimental.pallas import tpu as pltpu
from jax.experimental.pallas import tpu_sc as plsc

N = 2048
D = 768
T = 8
NC, NS, L = 2, 16, 16
NW = NC * NS
CHUNK = N // NW
BM = 128
NB = N // BM

_MESH = plsc.VectorSubcoreMesh(
    core_axis_name="c", subcore_axis_name="s", num_cores=NC, num_subcores=NS)


@functools.partial(
    pl.kernel,
    out_type=(
        jax.ShapeDtypeStruct((N, D), jnp.float32),
        jax.ShapeDtypeStruct((N,), jnp.int32),
        jax.ShapeDtypeStruct((L,), jnp.int32),
    ),
    mesh=_MESH,
    scratch_types=[
        pltpu.VMEM((N,), jnp.int32),
        pltpu.VMEM((CHUNK,), jnp.int32),
        pltpu.VMEM((CHUNK, D), jnp.float32),
        pltpu.VMEM((L,), jnp.int32),
        pltpu.SemaphoreType.DMA,
    ],
    compiler_params=pltpu.CompilerParams(needs_layout_passes=False),
)
def _route_kernel(x_hbm, xt_hbm, xs_hbm, pos_hbm, off_hbm,
                  ty_v, pos_v, rows_v, off_v, sem):
    w = lax.axis_index("s") * NC + lax.axis_index("c")
    base = w * CHUNK
    iota16 = lax.broadcasted_iota(jnp.int32, (L,), 0)

    pltpu.sync_copy(xt_hbm, ty_v)

    total = jnp.zeros((L,), jnp.int32)
    mine = jnp.zeros((L,), jnp.int32)
    for ci in range(NW):
        hc = jnp.zeros((L,), jnp.int32)
        for sub in range(CHUNK // L):
            tyc = ty_v[pl.ds(ci * CHUNK + sub * L, L)]
            for t in range(T):
                pc = jnp.sum((tyc == t).astype(jnp.int32))
                hc = jnp.where(iota16 == t, hc + jnp.full((L,), pc), hc)
        flag = jnp.full((L,), (ci < w).astype(jnp.int32))
        mine = mine + hc * flag
        total = total + hc
    cs = plsc.cumsum(total)
    excl = cs - total
    base_vec = excl + mine

    for ci in range(CHUNK // L):
        tyc = ty_v[pl.ds(base + ci * L, L)]
        posc = jnp.zeros((L,), jnp.int32)
        for t in range(T):
            m = tyc == t
            cm = plsc.cumsum(m.astype(jnp.int32))
            bt = jnp.sum(jnp.where(iota16 == t, base_vec, 0))
            posc = jnp.where(m, jnp.full((L,), bt) + cm - 1, posc)
            cnt = jnp.max(cm)
            base_vec = jnp.where(iota16 == t, base_vec + jnp.full((L,), cnt),
                                 base_vec)
        posc = jnp.minimum(jnp.maximum(posc, jnp.zeros((L,), jnp.int32)),
                           jnp.full((L,), N - 1))
        pos_v[pl.ds(ci * L, L)] = posc

    pltpu.sync_copy(pos_v, pos_hbm.at[pl.ds(base, CHUNK)])
    pltpu.sync_copy(x_hbm.at[pl.ds(base, CHUNK)], rows_v)
    pltpu.async_copy(rows_v, xs_hbm.at[pos_v], sem).wait()

    @pl.when(w == 0)
    def _offsets():
        off_v[...] = excl
        pltpu.sync_copy(off_v, off_hbm)


def _gemm_body(off_ref, x_ref, w_ref, b_ref, o_ref):
    t = pl.program_id(0)
    ri = lax.broadcasted_iota(jnp.int32, (BM, 1), 0)
    seg_s = off_ref[t]
    seg_e = off_ref[t + 1]
    for b in range(NB):
        lo = b * BM
        s = seg_s - lo
        e = seg_e - lo

        @pl.when((e > s) & (e > 0) & (s < BM))
        def _acc():
            prod = jnp.dot(x_ref[pl.ds(lo, BM), :], w_ref[0],
                           preferred_element_type=jnp.float32)
            prod = prod + b_ref[t][None, :]
            m = (ri >= s) & (ri < e)
            o_ref[pl.ds(lo, BM), :] = jnp.where(m, prod,
                                                o_ref[pl.ds(lo, BM), :])


def _grouped_gemm(offs, xs, W, b):
    grid_spec = pltpu.PrefetchScalarGridSpec(
        num_scalar_prefetch=1,
        grid=(T,),
        in_specs=[
            pl.BlockSpec((N, D), lambda g, off: (0, 0)),
            pl.BlockSpec((1, D, D), lambda g, off: (g, 0, 0)),
            pl.BlockSpec((T, D), lambda g, off: (0, 0)),
        ],
        out_specs=pl.BlockSpec((N, D), lambda g, off: (0, 0)),
    )
    return pl.pallas_call(
        _gemm_body,
        grid_spec=grid_spec,
        out_shape=jax.ShapeDtypeStruct((N, D), jnp.float32),
    )(offs, xs, W, b)


@functools.partial(
    pl.kernel,
    out_type=jax.ShapeDtypeStruct((N, D), jnp.float32),
    mesh=_MESH,
    scratch_types=[
        pltpu.VMEM((CHUNK,), jnp.int32),
        pltpu.VMEM((CHUNK, D), jnp.float32),
        pltpu.SemaphoreType.DMA,
    ],
    compiler_params=pltpu.CompilerParams(needs_layout_passes=False),
)
def _unsort_kernel(y_hbm, pos_hbm, out_hbm, pos_v, rows_v, sem):
    w = lax.axis_index("s") * NC + lax.axis_index("c")
    base = w * CHUNK
    pltpu.sync_copy(pos_hbm.at[pl.ds(base, CHUNK)], pos_v)
    pltpu.async_copy(y_hbm.at[pos_v], rows_v, sem).wait()
    pltpu.sync_copy(rows_v, out_hbm.at[pl.ds(base, CHUNK)])


@jax.jit
def kernel(x, x_type, W, b):
    xt = x_type.astype(jnp.int32)
    xs, pos, offs = _route_kernel(x, xt)
    y = _grouped_gemm(offs, xs, W, b)
    return _unsort_kernel(y, pos)
